# baseline (speedup 1.0000x reference)
import functools
import math

import jax
import jax.numpy as jnp
from jax import lax
from jax.experimental import pallas as pl
from jax.experimental.pallas import tpu as pltpu
from jax.experimental.pallas import tpu_sc as plsc

W_S = 4
N_W = W_S * W_S
B, L, D = 32, 577, 768
CLS = L // 2
H = 24
HW = 6
NQ = 2 * L
QPT = 36
NK = 42


def _build_rot(r, epsilon=1e-08):
    norms = jnp.linalg.norm(r, axis=1, keepdims=True)
    r = r / (norms + epsilon)
    angles = jnp.arctan2(r[:, 0], r[:, 1])
    cos = jnp.cos(angles)
    sin = jnp.sin(angles)
    row0 = jnp.stack([cos, -sin], axis=-1)
    row1 = jnp.stack([sin, cos], axis=-1)
    return jnp.stack([row0, row1], axis=1)


def _row_tables(scale, rotation, mean_p):
    scale_e = jnp.exp(scale)
    left = jax.vmap(jnp.diag)(scale_e)
    right = _build_rot(rotation)
    transform = left @ right
    cov = transform @ jnp.swapaxes(transform, -2, -1)
    chol = jnp.linalg.cholesky(cov)
    inv_cov = jax.vmap(
        lambda c: jax.scipy.linalg.cho_solve((c, True), jnp.eye(2, dtype=c.dtype))
    )(chol)
    grid_y, grid_x = jnp.meshgrid(
        jnp.arange(HW, dtype=jnp.float32),
        jnp.arange(HW, dtype=jnp.float32),
        indexing="ij",
    )
    grid = jnp.stack([grid_x, grid_y], axis=-1)
    mean = jnp.exp(mean_p)
    mean_mean = jnp.mean(mean, axis=1, keepdims=True)
    mean_std = jnp.std(mean, axis=1, keepdims=True, ddof=1)
    mean = (mean - mean_mean) / (mean_std + 1e-05)
    mean = mean * (HW // 2) + HW // 2
    mean = jnp.clip(mean, 0.0, float(HW // 2))
    diff = grid[None, :, :, :] - mean[:, None, None, :]
    maha = jnp.einsum("nhwi,nij,nhwj->nhw", diff, inv_cov, diff)
    weights = jax.nn.sigmoid(jnp.exp(-0.5 * maha)).reshape(N_W, HW * HW)
    iota36 = lax.broadcasted_iota(jnp.int32, (N_W, HW * HW), 1)
    _, idx, sorted_w = lax.sort(
        (-weights, iota36, weights), dimension=1, num_keys=1, is_stable=True
    )

    n = jnp.arange(N_W)[:, None]
    wy, wx = n // W_S, n % W_S
    sy, sx = idx // HW, idx % HW
    rs = (wy * HW + sy) * H + wx * HW + sx
    srow = jnp.transpose(
        rs.astype(jnp.int32).reshape(W_S, W_S, HW, HW), (0, 2, 1, 3)
    ).reshape(H * H)
    wrow = jnp.transpose(
        sorted_w.reshape(W_S, W_S, HW, HW), (0, 2, 1, 3)
    ).reshape(H * H)
    src_sp = srow + (srow >= CLS).astype(jnp.int32)
    src = jnp.concatenate(
        [src_sp[:CLS], jnp.array([CLS], jnp.int32), src_sp[CLS:]]
    )
    wgt = jnp.concatenate(
        [wrow[:CLS], jnp.array([1.0], jnp.float32), wrow[CLS:]]
    )
    return src, wgt


def _tile_tables(src, wgt):
    s2q = (src * 2)[:, None] + jnp.arange(2, dtype=jnp.int32)[None, :]
    s2q = s2q.reshape(NQ)
    w2q = jnp.broadcast_to(wgt[:, None], (L, 2)).reshape(NQ)
    main_s = s2q[: 32 * QPT].reshape(32, QPT)
    main_w = w2q[: 32 * QPT].reshape(32, QPT)
    extra_s = jnp.concatenate(
        [s2q[32 * QPT:], jnp.zeros((30,), jnp.int32)]
    ).reshape(32, 1)
    extra_w = jnp.concatenate(
        [w2q[32 * QPT:], jnp.zeros((30,), jnp.float32)]
    ).reshape(32, 1)
    s2full = jnp.concatenate(
        [main_s, extra_s, jnp.zeros((32, 64 - QPT - 1), jnp.int32)], axis=1
    )
    wtbl = jnp.concatenate(
        [main_w, extra_w, jnp.zeros((32, 128 - QPT - 1), jnp.float32)], axis=1
    ).reshape(32, 1, 128)
    idx8 = jnp.concatenate(
        [s2full[..., None], jnp.zeros((32, 64, 7), jnp.int32)], axis=2
    ).reshape(32, 1, 512)
    return idx8, wtbl


def _sc_body(x_hbm, idx_hbm, wtbl_hbm, out_hbm,
             idx_v, wtb_v, wtab, in0, in1, out0, out1, out2,
             gsem0, gsem1, ssem0, ssem1, ssem2):
    info = plsc.get_sparse_core_info()
    nc = info.num_cores
    wid = lax.axis_index("s") * nc + lax.axis_index("c")
    nvalid = QPT + jnp.where(wid < 2, 1, 0)

    ins = (in0, in1)
    outs = (out0, out1, out2)
    gsems = (gsem0, gsem1)
    ssems = (ssem0, ssem1, ssem2)

    pltpu.sync_copy(idx_hbm.at[wid], idx_v)
    pltpu.sync_copy(wtbl_hbm.at[wid], wtb_v)
    for j in range(3):
        wv16 = wtb_v[0, pl.ds(16 * j, 16)]
        for r in range(16):
            wtab[16 * j + r, :] = jnp.full((16,), wv16[r], jnp.float32)

    def start_gather(kk, p):
        off = pl.multiple_of(8 * kk, 8)
        pltpu.async_copy(x_hbm.at[idx_v.at[0, pl.ds(off, 1)]], ins[p], gsems[p])

    def wait_gather(p):
        pltpu.make_async_copy(
            x_hbm.at[idx_v.at[0, pl.ds(0, 1)]], ins[p], gsems[p]).wait()

    def wait_scatter(p):
        pltpu.make_async_copy(
            outs[p], out_hbm.at[pl.ds(0, 1)], ssems[p]).wait()

    def scale(src_buf, dst_buf, kk):
        wv = wtab[kk, :]

        def body(r, _):
            for c in range(D // 16):
                cs = pl.ds(c * 16, 16)
                dst_buf[0, r, cs] = src_buf[0, r, cs] * wv
            return 0

        lax.fori_loop(0, 16, body, 0)

    start_gather(0, 0)
    start_gather(1, 1)

    def six(g, _):
        for bb in range(6):
            kk = g * 6 + bb
            pin = bb % 2
            pout = bb % 3
            wait_gather(pin)

            @pl.when((kk >= 3) & (kk - 3 < nvalid))
            def _():
                wait_scatter(pout)

            @pl.when(kk < nvalid)
            def _():
                scale(ins[pin], outs[pout], kk)
                qo = jnp.where(kk == QPT, 1152 + wid, wid * QPT + kk)
                pltpu.async_copy(outs[pout], out_hbm.at[pl.ds(qo, 1)], ssems[pout])

            start_gather(kk + 2, pin)
        return 0

    lax.fori_loop(0, NK // 6, six, 0)

    for bb in range(2):
        wait_gather(bb)


@jax.jit
def kernel(x, scale, rotation, mean):
    src, wgt = _row_tables(scale, rotation, mean)
    idx8, wtbl = _tile_tables(src, wgt)
    x2 = jnp.transpose(x, (1, 0, 2)).reshape(NQ, 16, D)

    mesh = plsc.VectorSubcoreMesh(core_axis_name="c", subcore_axis_name="s")
    run = functools.partial(
        pl.kernel,
        mesh=mesh,
        out_type=jax.ShapeDtypeStruct((NQ, 16, D), jnp.float32),
        scratch_types=[
            pltpu.VMEM((1, 512), jnp.int32),
            pltpu.VMEM((1, 128), jnp.float32),
            pltpu.VMEM((48, 16), jnp.float32),
            pltpu.VMEM((1, 16, D), jnp.float32),
            pltpu.VMEM((1, 16, D), jnp.float32),
            pltpu.VMEM((1, 16, D), jnp.float32),
            pltpu.VMEM((1, 16, D), jnp.float32),
            pltpu.VMEM((1, 16, D), jnp.float32),
            pltpu.SemaphoreType.DMA,
            pltpu.SemaphoreType.DMA,
            pltpu.SemaphoreType.DMA,
            pltpu.SemaphoreType.DMA,
            pltpu.SemaphoreType.DMA,
        ],
    )(_sc_body)
    out2 = run(x2, idx8, wtbl)
    return jnp.transpose(out2.reshape(L, B, D), (1, 0, 2))


# speedup vs baseline: 1.1286x; 1.1286x over previous
import functools
import math

import jax
import jax.numpy as jnp
from jax import lax
from jax.experimental import pallas as pl
from jax.experimental.pallas import tpu as pltpu
from jax.experimental.pallas import tpu_sc as plsc

W_S = 4
N_W = W_S * W_S
B, L, D = 32, 577, 768
CLS = L // 2
H = 24
HW = 6
NQ = 2 * L
QPT = 36
NK = 38


def _build_rot(r, epsilon=1e-08):
    norms = jnp.linalg.norm(r, axis=1, keepdims=True)
    r = r / (norms + epsilon)
    angles = jnp.arctan2(r[:, 0], r[:, 1])
    cos = jnp.cos(angles)
    sin = jnp.sin(angles)
    row0 = jnp.stack([cos, -sin], axis=-1)
    row1 = jnp.stack([sin, cos], axis=-1)
    return jnp.stack([row0, row1], axis=1)


def _row_tables(scale, rotation, mean_p):
    scale_e = jnp.exp(scale)
    left = jax.vmap(jnp.diag)(scale_e)
    right = _build_rot(rotation)
    transform = left @ right
    cov = transform @ jnp.swapaxes(transform, -2, -1)
    chol = jnp.linalg.cholesky(cov)
    inv_cov = jax.vmap(
        lambda c: jax.scipy.linalg.cho_solve((c, True), jnp.eye(2, dtype=c.dtype))
    )(chol)
    grid_y, grid_x = jnp.meshgrid(
        jnp.arange(HW, dtype=jnp.float32),
        jnp.arange(HW, dtype=jnp.float32),
        indexing="ij",
    )
    grid = jnp.stack([grid_x, grid_y], axis=-1)
    mean = jnp.exp(mean_p)
    mean_mean = jnp.mean(mean, axis=1, keepdims=True)
    mean_std = jnp.std(mean, axis=1, keepdims=True, ddof=1)
    mean = (mean - mean_mean) / (mean_std + 1e-05)
    mean = mean * (HW // 2) + HW // 2
    mean = jnp.clip(mean, 0.0, float(HW // 2))
    diff = grid[None, :, :, :] - mean[:, None, None, :]
    maha = jnp.einsum("nhwi,nij,nhwj->nhw", diff, inv_cov, diff)
    weights = jax.nn.sigmoid(jnp.exp(-0.5 * maha)).reshape(N_W, HW * HW)
    iota36 = lax.broadcasted_iota(jnp.int32, (N_W, HW * HW), 1)
    _, idx, sorted_w = lax.sort(
        (-weights, iota36, weights), dimension=1, num_keys=1, is_stable=True
    )

    n = jnp.arange(N_W)[:, None]
    wy, wx = n // W_S, n % W_S
    sy, sx = idx // HW, idx % HW
    rs = (wy * HW + sy) * H + wx * HW + sx
    srow = jnp.transpose(
        rs.astype(jnp.int32).reshape(W_S, W_S, HW, HW), (0, 2, 1, 3)
    ).reshape(H * H)
    wrow = jnp.transpose(
        sorted_w.reshape(W_S, W_S, HW, HW), (0, 2, 1, 3)
    ).reshape(H * H)
    src_sp = srow + (srow >= CLS).astype(jnp.int32)
    src = jnp.concatenate(
        [src_sp[:CLS], jnp.array([CLS], jnp.int32), src_sp[CLS:]]
    )
    wgt = jnp.concatenate(
        [wrow[:CLS], jnp.array([1.0], jnp.float32), wrow[CLS:]]
    )
    return src, wgt


def _tile_tables(src, wgt):
    s2q = (src * 2)[:, None] + jnp.arange(2, dtype=jnp.int32)[None, :]
    s2q = s2q.reshape(NQ)
    w2q = jnp.broadcast_to(wgt[:, None], (L, 2)).reshape(NQ)
    main_s = s2q[: 32 * QPT].reshape(32, QPT)
    main_w = w2q[: 32 * QPT].reshape(32, QPT)
    extra_s = jnp.concatenate(
        [s2q[32 * QPT:], jnp.zeros((30,), jnp.int32)]
    ).reshape(32, 1)
    extra_w = jnp.concatenate(
        [w2q[32 * QPT:], jnp.zeros((30,), jnp.float32)]
    ).reshape(32, 1)
    s2full = jnp.concatenate(
        [main_s, extra_s, jnp.zeros((32, 64 - QPT - 1), jnp.int32)], axis=1
    )
    wtbl = jnp.concatenate(
        [main_w, extra_w, jnp.zeros((32, 128 - QPT - 1), jnp.float32)], axis=1
    ).reshape(32, 1, 128)
    idx8 = jnp.concatenate(
        [s2full[..., None], jnp.zeros((32, 64, 7), jnp.int32)], axis=2
    ).reshape(32, 1, 512)
    return idx8, wtbl


def _sc_body(x_hbm, idx_hbm, wtbl_hbm, out_hbm,
             idx_v, wtb_v, wtab, in0, in1, out0, out1,
             gsem0, gsem1, ssem0, ssem1):
    info = plsc.get_sparse_core_info()
    nc = info.num_cores
    wid = lax.axis_index("s") * nc + lax.axis_index("c")
    nvalid = QPT + jnp.where(wid < 2, 1, 0)

    ins = (in0, in1)
    outs = (out0, out1)
    gsems = (gsem0, gsem1)
    ssems = (ssem0, ssem1)

    pltpu.sync_copy(idx_hbm.at[wid], idx_v)
    pltpu.sync_copy(wtbl_hbm.at[wid], wtb_v)
    for j in range(3):
        wv16 = wtb_v[0, pl.ds(16 * j, 16)]
        for r in range(16):
            wtab[16 * j + r, :] = jnp.full((16,), wv16[r], jnp.float32)

    def start_gather(kk, p):
        off = pl.multiple_of(8 * kk, 8)
        pltpu.async_copy(x_hbm.at[idx_v.at[0, pl.ds(off, 1)]], ins[p], gsems[p])

    def wait_gather(p):
        pltpu.make_async_copy(
            x_hbm.at[idx_v.at[0, pl.ds(0, 1)]], ins[p], gsems[p]).wait()

    def wait_scatter(p):
        pltpu.make_async_copy(
            outs[p], out_hbm.at[pl.ds(0, 1)], ssems[p]).wait()

    def scale(src_buf, dst_buf, kk):
        wv = wtab[kk, :]

        def body(r, _):
            for c in range(D // 16):
                cs = pl.ds(c * 16, 16)
                dst_buf[0, r, cs] = src_buf[0, r, cs] * wv
            return 0

        lax.fori_loop(0, 16, body, 0)

    start_gather(0, 0)
    start_gather(1, 1)

    def pair(g, _):
        for bb in range(2):
            kk = g * 2 + bb
            wait_gather(bb)

            @pl.when((kk >= 2) & (kk - 2 < nvalid))
            def _():
                wait_scatter(bb)

            @pl.when(kk < nvalid)
            def _():
                scale(ins[bb], outs[bb], kk)
                qo = jnp.where(kk == QPT, 1152 + wid, wid * QPT + kk)
                pltpu.async_copy(outs[bb], out_hbm.at[pl.ds(qo, 1)], ssems[bb])

            start_gather(kk + 2, bb)
        return 0

    lax.fori_loop(0, NK // 2, pair, 0)

    for bb in range(2):
        wait_gather(bb)

        @pl.when(NK - 2 + bb < nvalid)
        def _():
            wait_scatter(bb)


@jax.jit
def kernel(x, scale, rotation, mean):
    src, wgt = _row_tables(scale, rotation, mean)
    idx8, wtbl = _tile_tables(src, wgt)
    x2 = jnp.transpose(x, (1, 0, 2)).reshape(NQ, 16, D)

    mesh = plsc.VectorSubcoreMesh(core_axis_name="c", subcore_axis_name="s")
    run = functools.partial(
        pl.kernel,
        mesh=mesh,
        out_type=jax.ShapeDtypeStruct((NQ, 16, D), jnp.float32),
        scratch_types=[
            pltpu.VMEM((1, 512), jnp.int32),
            pltpu.VMEM((1, 128), jnp.float32),
            pltpu.VMEM((48, 16), jnp.float32),
            pltpu.VMEM((1, 16, D), jnp.float32),
            pltpu.VMEM((1, 16, D), jnp.float32),
            pltpu.VMEM((1, 16, D), jnp.float32),
            pltpu.VMEM((1, 16, D), jnp.float32),
            pltpu.SemaphoreType.DMA,
            pltpu.SemaphoreType.DMA,
            pltpu.SemaphoreType.DMA,
            pltpu.SemaphoreType.DMA,
        ],
    )(_sc_body)
    out2 = run(x2, idx8, wtbl)
    return jnp.transpose(out2.reshape(L, B, D), (1, 0, 2))
